# Initial kernel scaffold; baseline (speedup 1.0000x reference)
import jax
import jax.numpy as jnp
from jax.experimental import pallas as pl


def kernel(x, edge_index, W1_l, b1, W1_r, W2_l, b2, W2_r):
    raise NotImplementedError("write your pallas kernel here")


# speedup vs baseline: 10.8723x; 10.8723x over previous
import functools

import jax
import jax.numpy as jnp
from jax import lax
from jax.experimental import pallas as pl
from jax.experimental.pallas import tpu as pltpu
from jax.experimental.pallas import tpu_sc as plsc

_N = 10000
_E = 320000
_D_IN = 128
_D_HID = 1024
_D_OUT = 128

_NCORES = 2
_NSUB = 16
_NTILES = _NCORES * _NSUB
_CHUNK = 80

_ROWS_BLK = 1000


_N_PAD = 10240
_D = 128


def _make_seg_sum(with_counts):
    rows_per_tile = _N_PAD // _NSUB
    edges_per_tile = _E // _NTILES
    n_iter = edges_per_tile // _CHUNK
    mesh = plsc.VectorSubcoreMesh(core_axis_name="c", subcore_axis_name="s")

    out_type = [jax.ShapeDtypeStruct((_NCORES, _N_PAD, _D), jnp.float32)]
    scratch = [
        pltpu.VMEM((_CHUNK,), jnp.int32),
        pltpu.VMEM((_CHUNK,), jnp.int32),
        pltpu.VMEM((_CHUNK, _D), jnp.float32),
        pltpu.VMEM_SHARED((_N_PAD, _D), jnp.float32),
        pltpu.SemaphoreType.DMA,
    ]
    if with_counts:
        out_type.append(jax.ShapeDtypeStruct((_NTILES, _N_PAD), jnp.float32))
        scratch.append(pltpu.VMEM((_N_PAD,), jnp.float32))

    def body(tbl, src_h, dst_h, zer, zer1, out, out_cnt,
             src_v, dst_v, rows_v, acc, sem, hist_v):
        cid = lax.axis_index("c")
        sid = lax.axis_index("s")
        wid = sid * _NCORES + cid
        my_rows = pl.ds(sid * rows_per_tile, rows_per_tile)
        pltpu.sync_copy(zer.at[my_rows], acc.at[my_rows])
        if with_counts:
            pltpu.sync_copy(zer1, hist_v)
        plsc.subcore_barrier()
        ones16 = jnp.ones((16,), jnp.float32)

        def step(i, carry):
            base = wid * edges_per_tile + i * _CHUNK
            pltpu.sync_copy(src_h.at[pl.ds(base, _CHUNK)], src_v)
            pltpu.sync_copy(dst_h.at[pl.ds(base, _CHUNK)], dst_v)
            pltpu.async_copy(tbl.at[src_v], rows_v, sem).wait()
            pltpu.sync_copy(rows_v, acc.at[dst_v], add=True)
            if with_counts:
                for k in range(_CHUNK // 16):
                    idx = dst_v[pl.ds(k * 16, 16)]
                    plsc.addupdate_scatter(hist_v, [idx], ones16)
            return carry

        lax.fori_loop(0, n_iter, step, 0)
        plsc.subcore_barrier()
        pltpu.sync_copy(acc.at[my_rows], out.at[cid, my_rows])
        if with_counts:
            pltpu.sync_copy(hist_v, out_cnt.at[wid])

    if with_counts:
        def body_c(tbl, src_h, dst_h, zer, zer1, out, out_cnt,
                   src_v, dst_v, rows_v, acc, sem, hist_v):
            body(tbl, src_h, dst_h, zer, zer1, out, out_cnt,
                 src_v, dst_v, rows_v, acc, sem, hist_v)
        fn = body_c
    else:
        def body_n(tbl, src_h, dst_h, zer, out,
                   src_v, dst_v, rows_v, acc, sem):
            body(tbl, src_h, dst_h, zer, None, out, None,
                 src_v, dst_v, rows_v, acc, sem, None)
        fn = body_n

    return pl.kernel(
        fn, out_type=out_type, mesh=mesh, scratch_types=scratch,
        compiler_params=pltpu.CompilerParams(needs_layout_passes=False))


_seg_sum_cache = {}


def _seg_sum(with_counts):
    if with_counts not in _seg_sum_cache:
        _seg_sum_cache[with_counts] = _make_seg_sum(with_counts)
    return _seg_sum_cache[with_counts]


def _l1_body(x_r, a0_r, a1_r, c_r, w1l_r, b1_r, w1r_r, w2l_r, w2r_r,
             p_r, q_r):
    cnt = jnp.sum(c_r[...], axis=1, keepdims=True)
    inv = 1.0 / jnp.maximum(cnt, 1.0)
    agg = (a0_r[...] + a1_r[...]) * inv
    h = agg @ w1l_r[...] + b1_r[...] + x_r[...] @ w1r_r[...]
    h = jnp.maximum(h, 0.0)
    p_r[...] = h @ w2l_r[...]
    q_r[...] = h @ w2r_r[...]


def _layer1_fused(x, a0, a1, cnt_t, w1l, b1, w1r, w2l, w2r):
    nb = _N // _ROWS_BLK
    row_spec = lambda w: pl.BlockSpec((_ROWS_BLK, w), lambda i: (i, 0))
    full_spec = lambda r, c: pl.BlockSpec((r, c), lambda i: (0, 0))
    return pl.pallas_call(
        _l1_body,
        grid=(nb,),
        in_specs=[
            row_spec(_D_IN), row_spec(_D_IN), row_spec(_D_IN),
            row_spec(_NTILES),
            full_spec(_D_IN, _D_HID), full_spec(1, _D_HID),
            full_spec(_D_IN, _D_HID),
            full_spec(_D_HID, _D_OUT), full_spec(_D_HID, _D_OUT),
        ],
        out_specs=[row_spec(_D_OUT), row_spec(_D_OUT)],
        out_shape=[
            jax.ShapeDtypeStruct((_N, _D_OUT), jnp.float32),
            jax.ShapeDtypeStruct((_N, _D_OUT), jnp.float32),
        ],
    )(x, a0, a1, cnt_t, w1l, b1.reshape(1, _D_HID), w1r, w2l, w2r)


def _l2_body(a0_r, a1_r, c_r, q_r, b2_r, out_r):
    cnt = jnp.sum(c_r[...], axis=1, keepdims=True)
    inv = 1.0 / jnp.maximum(cnt, 1.0)
    o = (a0_r[...] + a1_r[...]) * inv + b2_r[...] + q_r[...]
    m = jnp.max(o, axis=1, keepdims=True)
    s = jnp.sum(jnp.exp(o - m), axis=1, keepdims=True)
    out_r[...] = o - m - jnp.log(s)


def _layer2_final(a0, a1, cnt_t, q, b2):
    nb = _N // _ROWS_BLK
    row_spec = lambda w: pl.BlockSpec((_ROWS_BLK, w), lambda i: (i, 0))
    return pl.pallas_call(
        _l2_body,
        grid=(nb,),
        in_specs=[
            row_spec(_D_OUT), row_spec(_D_OUT),
            row_spec(_NTILES),
            row_spec(_D_OUT),
            pl.BlockSpec((1, _D_OUT), lambda i: (0, 0)),
        ],
        out_specs=row_spec(_D_OUT),
        out_shape=jax.ShapeDtypeStruct((_N, _D_OUT), jnp.float32),
    )(a0, a1, cnt_t, q, b2.reshape(1, _D_OUT))


def kernel(x, edge_index, W1_l, b1, W1_r, W2_l, b2, W2_r):
    src = edge_index[0]
    dst = edge_index[1]

    zer = jnp.zeros((_N_PAD, _D), jnp.float32)
    zer1 = jnp.zeros((_N_PAD,), jnp.float32)
    agg1, cnt = _seg_sum(True)(x, src, dst, zer, zer1)
    cnt_t = cnt.T[:_N]
    a0, a1 = agg1[0, :_N], agg1[1, :_N]

    p, q = _layer1_fused(x, a0, a1, cnt_t, W1_l, b1, W1_r, W2_l, W2_r)

    (agg2,) = _seg_sum(False)(p, src, dst, zer)

    return _layer2_final(agg2[0, :_N], agg2[1, :_N], cnt_t, q, b2)


# speedup vs baseline: 23.0592x; 2.1209x over previous
import functools

import jax
import jax.numpy as jnp
from jax import lax
from jax.experimental import pallas as pl
from jax.experimental.pallas import tpu as pltpu
from jax.experimental.pallas import tpu_sc as plsc

_N = 10000
_E = 320000
_D_IN = 128
_D_HID = 1024
_D_OUT = 128

_NCORES = 2
_NSUB = 16
_NTILES = _NCORES * _NSUB
_CHUNK = 80

_ROWS_BLK = 1000


_N_PAD = 10240
_D = 128


_NBUF = 2


def _make_seg_sum(with_counts):
    rows_per_tile = _N_PAD // _NSUB
    edges_per_tile = _E // _NTILES
    n_iter = edges_per_tile // _CHUNK
    mesh = plsc.VectorSubcoreMesh(core_axis_name="c", subcore_axis_name="s")

    out_type = [jax.ShapeDtypeStruct((_NCORES, _N_PAD, _D), jnp.float32)]
    scratch = (
        [pltpu.VMEM((_CHUNK,), jnp.int32)] * _NBUF
        + [pltpu.VMEM((_CHUNK,), jnp.int32)] * _NBUF
        + [pltpu.VMEM((_CHUNK, _D), jnp.float32)] * _NBUF
        + [pltpu.VMEM_SHARED((_N_PAD, _D), jnp.float32)]
        + [pltpu.SemaphoreType.DMA] * (4 * _NBUF))
    if with_counts:
        out_type.append(jax.ShapeDtypeStruct((_NTILES, _N_PAD), jnp.float32))
        scratch.append(pltpu.VMEM((_N_PAD,), jnp.float32))

    def body(tbl, src_h, dst_h, zer, zer1, out, out_cnt,
             src_v, dst_v, rows_v, acc, sems, hist_v):
        cid = lax.axis_index("c")
        sid = lax.axis_index("s")
        wid = sid * _NCORES + cid
        my_rows = pl.ds(sid * rows_per_tile, rows_per_tile)
        sem_g = sems[0 * _NBUF:1 * _NBUF]
        sem_s = sems[1 * _NBUF:2 * _NBUF]
        sem_si = sems[2 * _NBUF:3 * _NBUF]
        sem_di = sems[3 * _NBUF:4 * _NBUF]
        e0 = wid * edges_per_tile
        pltpu.sync_copy(zer.at[my_rows], acc.at[my_rows])
        if with_counts:
            pltpu.sync_copy(zer1, hist_v)
        plsc.subcore_barrier()
        ones16 = jnp.ones((16,), jnp.float32)

        def _idx(which, ring, sem, i, b):
            return (which.at[pl.ds(e0 + i * _CHUNK, _CHUNK)], ring[b], sem[b])

        def si_issue(i, b):
            pltpu.async_copy(*_idx(src_h, src_v, sem_si, i, b))

        def si_wait(i, b):
            pltpu.make_async_copy(*_idx(src_h, src_v, sem_si, i, b)).wait()

        def di_issue(i, b):
            pltpu.async_copy(*_idx(dst_h, dst_v, sem_di, i, b))

        def di_wait(i, b):
            pltpu.make_async_copy(*_idx(dst_h, dst_v, sem_di, i, b)).wait()

        def gather(i, b):
            pltpu.async_copy(tbl.at[src_v[b]], rows_v[b], sem_g[b])

        def gather_wait(i, b):
            pltpu.make_async_copy(tbl.at[src_v[b]], rows_v[b],
                                  sem_g[b]).wait()

        def scat(i, b):
            pltpu.async_copy(rows_v[b], acc.at[dst_v[b]], sem_s[b], add=True)

        def scat_wait(i, b):
            pltpu.make_async_copy(rows_v[b], acc.at[dst_v[b]],
                                  sem_s[b]).wait()

        def stage(i, b, g, first, last):
            bn = 1 - b
            if first:
                @pl.when(g > 0)
                def _():
                    scat_wait(i - 1, bn)
            else:
                scat_wait(i - 1, bn)
            if not last:
                di_issue(i + 1, bn)
                si_wait(i + 1, bn)
                gather(i + 1, bn)
            gather_wait(i, b)
            if not last:
                if b == 0:
                    si_issue(i + 2, b)
                else:
                    @pl.when(g < (n_iter - 1) // _NBUF - 1)
                    def _():
                        si_issue(i + 2, b)
            if first:
                @pl.when(g > 0)
                def _():
                    di_wait(i, b)
            else:
                di_wait(i, b)
            scat(i, b)
            if with_counts:
                for k in range(_CHUNK // 16):
                    idx = dst_v[b][pl.ds(k * 16, 16)]
                    plsc.addupdate_scatter(hist_v, [idx], ones16)

        pltpu.sync_copy(src_h.at[pl.ds(e0, _CHUNK)], src_v[0])
        pltpu.sync_copy(dst_h.at[pl.ds(e0, _CHUNK)], dst_v[0])
        gather(0, 0)
        si_issue(1, 1)

        def group(g, carry):
            stage(2 * g, 0, g, True, False)
            stage(2 * g + 1, 1, g, False, False)
            return carry

        lax.fori_loop(0, n_iter // 2, group, 0)
        stage(n_iter - 1, 0, None, False, True)
        scat_wait(n_iter - 1, 0)
        plsc.subcore_barrier()
        pltpu.sync_copy(acc.at[my_rows], out.at[cid, my_rows])
        if with_counts:
            pltpu.sync_copy(hist_v, out_cnt.at[wid])

    def _split(scr):
        b = _NBUF
        return (list(scr[0:b]), list(scr[b:2 * b]), list(scr[2 * b:3 * b]),
                scr[3 * b], list(scr[3 * b + 1:7 * b + 1]))

    if with_counts:
        def body_c(tbl, src_h, dst_h, zer, zer1, out, out_cnt, *scr):
            sv, dv, rv, acc, sems = _split(scr)
            body(tbl, src_h, dst_h, zer, zer1, out, out_cnt,
                 sv, dv, rv, acc, sems, scr[-1])
        fn = body_c
    else:
        def body_n(tbl, src_h, dst_h, zer, out, *scr):
            sv, dv, rv, acc, sems = _split(scr)
            body(tbl, src_h, dst_h, zer, None, out, None,
                 sv, dv, rv, acc, sems, None)
        fn = body_n

    return pl.kernel(
        fn, out_type=out_type, mesh=mesh, scratch_types=scratch,
        compiler_params=pltpu.CompilerParams(needs_layout_passes=False))


_seg_sum_cache = {}


def _seg_sum(with_counts):
    if with_counts not in _seg_sum_cache:
        _seg_sum_cache[with_counts] = _make_seg_sum(with_counts)
    return _seg_sum_cache[with_counts]


def _l1_body(x_r, a0_r, a1_r, c_r, w1l_r, b1_r, w1r_r, w2l_r, w2r_r,
             p_r, q_r):
    cnt = jnp.sum(c_r[...], axis=1, keepdims=True)
    inv = 1.0 / jnp.maximum(cnt, 1.0)
    agg = (a0_r[...] + a1_r[...]) * inv
    h = agg @ w1l_r[...] + b1_r[...] + x_r[...] @ w1r_r[...]
    h = jnp.maximum(h, 0.0)
    p_r[...] = h @ w2l_r[...]
    q_r[...] = h @ w2r_r[...]


def _layer1_fused(x, a0, a1, cnt_t, w1l, b1, w1r, w2l, w2r):
    nb = _N // _ROWS_BLK
    row_spec = lambda w: pl.BlockSpec((_ROWS_BLK, w), lambda i: (i, 0))
    full_spec = lambda r, c: pl.BlockSpec((r, c), lambda i: (0, 0))
    return pl.pallas_call(
        _l1_body,
        grid=(nb,),
        in_specs=[
            row_spec(_D_IN), row_spec(_D_IN), row_spec(_D_IN),
            row_spec(_NTILES),
            full_spec(_D_IN, _D_HID), full_spec(1, _D_HID),
            full_spec(_D_IN, _D_HID),
            full_spec(_D_HID, _D_OUT), full_spec(_D_HID, _D_OUT),
        ],
        out_specs=[row_spec(_D_OUT), row_spec(_D_OUT)],
        out_shape=[
            jax.ShapeDtypeStruct((_N, _D_OUT), jnp.float32),
            jax.ShapeDtypeStruct((_N, _D_OUT), jnp.float32),
        ],
    )(x, a0, a1, cnt_t, w1l, b1.reshape(1, _D_HID), w1r, w2l, w2r)


def _l2_body(a0_r, a1_r, c_r, q_r, b2_r, out_r):
    cnt = jnp.sum(c_r[...], axis=1, keepdims=True)
    inv = 1.0 / jnp.maximum(cnt, 1.0)
    o = (a0_r[...] + a1_r[...]) * inv + b2_r[...] + q_r[...]
    m = jnp.max(o, axis=1, keepdims=True)
    s = jnp.sum(jnp.exp(o - m), axis=1, keepdims=True)
    out_r[...] = o - m - jnp.log(s)


def _layer2_final(a0, a1, cnt_t, q, b2):
    nb = _N // _ROWS_BLK
    row_spec = lambda w: pl.BlockSpec((_ROWS_BLK, w), lambda i: (i, 0))
    return pl.pallas_call(
        _l2_body,
        grid=(nb,),
        in_specs=[
            row_spec(_D_OUT), row_spec(_D_OUT),
            row_spec(_NTILES),
            row_spec(_D_OUT),
            pl.BlockSpec((1, _D_OUT), lambda i: (0, 0)),
        ],
        out_specs=row_spec(_D_OUT),
        out_shape=jax.ShapeDtypeStruct((_N, _D_OUT), jnp.float32),
    )(a0, a1, cnt_t, q, b2.reshape(1, _D_OUT))


def kernel(x, edge_index, W1_l, b1, W1_r, W2_l, b2, W2_r):
    src = edge_index[0]
    dst = edge_index[1]

    zer = jnp.zeros((_N_PAD, _D), jnp.float32)
    zer1 = jnp.zeros((_N_PAD,), jnp.float32)
    agg1, cnt = _seg_sum(True)(x, src, dst, zer, zer1)
    cnt_t = cnt.T[:_N]
    a0, a1 = agg1[0, :_N], agg1[1, :_N]

    p, q = _layer1_fused(x, a0, a1, cnt_t, W1_l, b1, W1_r, W2_l, W2_r)

    (agg2,) = _seg_sum(False)(p, src, dst, zer)

    return _layer2_final(agg2[0, :_N], agg2[1, :_N], cnt_t, q, b2)
